# Initial kernel scaffold; baseline (speedup 1.0000x reference)
import jax
import jax.numpy as jnp
from jax.experimental import pallas as pl


def kernel(x, edge_index, W1, b1, W2, b2, W3, b3, Wp, bp):
    raise NotImplementedError("write your pallas kernel here")


# speedup vs baseline: 14.6847x; 14.6847x over previous
import functools

import jax
import jax.numpy as jnp
from jax import lax
from jax.experimental import pallas as pl
from jax.experimental.pallas import tpu as pltpu
from jax.experimental.pallas import tpu_sc as plsc

N = 10000
E = 320000
NPAD = 10240
NCORES = 2
NSUB = 16
ROWS_PER_TILE = NPAD // (NCORES * NSUB) * NCORES
J = 8
R = 128
CHUNK = J * R
G = 10
EPAD = NCORES * NSUB * G * CHUNK


def _make_agg(D):
    mesh = plsc.VectorSubcoreMesh(core_axis_name="c", subcore_axis_name="s")

    @functools.partial(
        pl.kernel,
        out_type=jax.ShapeDtypeStruct((NCORES, NPAD, D), jnp.float32),
        mesh=mesh,
        scratch_types=[
            pltpu.VMEM((J, R), jnp.int32),
            pltpu.VMEM((J, R), jnp.int32),
            pltpu.VMEM((CHUNK, D), jnp.float32),
            pltpu.VMEM_SHARED((NPAD, D), jnp.float32),
            pltpu.SemaphoreType.DMA,
        ],
        compiler_params=pltpu.CompilerParams(use_tc_tiling_on_sc=False),
    )
    def agg(g_hbm, src_hbm, dst_hbm, zeros_hbm, out_hbm, sidx, didx, rows, acc, sem):
        cid = lax.axis_index("c")
        sid = lax.axis_index("s")
        rlo = sid * ROWS_PER_TILE
        pltpu.sync_copy(zeros_hbm, rows.at[pl.ds(0, ROWS_PER_TILE)])
        pltpu.sync_copy(rows.at[pl.ds(0, ROWS_PER_TILE)], acc.at[pl.ds(rlo, ROWS_PER_TILE)])
        plsc.subcore_barrier()

        def chunk_body(c, carry):
            pltpu.sync_copy(src_hbm.at[cid, sid, c], sidx)
            pltpu.sync_copy(dst_hbm.at[cid, sid, c], didx)
            descs = [
                pltpu.async_copy(g_hbm.at[sidx.at[j]], rows.at[pl.ds(j * R, R)], sem)
                for j in range(J)
            ]
            for d in descs:
                d.wait()
            for j in range(J):
                pltpu.sync_copy(rows.at[pl.ds(j * R, R)], acc.at[didx.at[j]], add=True)
            return carry

        lax.fori_loop(0, G, chunk_body, 0)
        plsc.subcore_barrier()
        pltpu.sync_copy(acc.at[pl.ds(rlo, ROWS_PER_TILE)], rows.at[pl.ds(0, ROWS_PER_TILE)])
        pltpu.sync_copy(rows.at[pl.ds(0, ROWS_PER_TILE)], out_hbm.at[cid, pl.ds(rlo, ROWS_PER_TILE)])

    return agg


_agg64 = _make_agg(64)
_agg32 = _make_agg(32)

_DEG_D = 16


def _make_deg():
    mesh = plsc.VectorSubcoreMesh(core_axis_name="c", subcore_axis_name="s")

    @functools.partial(
        pl.kernel,
        out_type=jax.ShapeDtypeStruct((NCORES, NPAD, _DEG_D), jnp.float32),
        mesh=mesh,
        scratch_types=[
            pltpu.VMEM((J, R), jnp.int32),
            pltpu.VMEM((CHUNK, _DEG_D), jnp.float32),
            pltpu.VMEM_SHARED((NPAD, _DEG_D), jnp.float32),
        ],
        compiler_params=pltpu.CompilerParams(use_tc_tiling_on_sc=False),
    )
    def deg(dst_hbm, zeros_hbm, ones_hbm, out_hbm, didx, rows, acc):
        cid = lax.axis_index("c")
        sid = lax.axis_index("s")
        rlo = sid * ROWS_PER_TILE
        pltpu.sync_copy(zeros_hbm, rows.at[pl.ds(0, ROWS_PER_TILE)])
        pltpu.sync_copy(rows.at[pl.ds(0, ROWS_PER_TILE)], acc.at[pl.ds(rlo, ROWS_PER_TILE)])
        plsc.subcore_barrier()
        pltpu.sync_copy(ones_hbm, rows)

        def chunk_body(c, carry):
            pltpu.sync_copy(dst_hbm.at[cid, sid, c], didx)
            for j in range(J):
                pltpu.sync_copy(rows.at[pl.ds(j * R, R)], acc.at[didx.at[j]], add=True)
            return carry

        lax.fori_loop(0, G, chunk_body, 0)
        plsc.subcore_barrier()
        pltpu.sync_copy(acc.at[pl.ds(rlo, ROWS_PER_TILE)], rows.at[pl.ds(0, ROWS_PER_TILE)])
        pltpu.sync_copy(rows.at[pl.ds(0, ROWS_PER_TILE)], out_hbm.at[cid, pl.ds(rlo, ROWS_PER_TILE)])

    return deg


_deg_kernel = _make_deg()


def _dinv_from(deg_ref):
    d = deg_ref[0, :N, 0:1] + deg_ref[1, :N, 0:1] + 1.0
    return lax.rsqrt(jnp.maximum(d, 1e-12))


def _prep_body(deg_ref, x_ref, w_ref, o_ref):
    dinv = _dinv_from(deg_ref)
    h = jnp.dot(x_ref[...], w_ref[...], preferred_element_type=jnp.float32)
    o_ref[...] = h * dinv


def _combine_body(deg_ref, s_ref, g_ref, b_ref, w_ref, o_ref):
    dinv = _dinv_from(deg_ref)
    s = s_ref[0, :N, :] + s_ref[1, :N, :] + g_ref[...]
    xn = jnp.maximum(dinv * s + b_ref[...], 0.0)
    o_ref[...] = jnp.dot(xn, w_ref[...], preferred_element_type=jnp.float32) * dinv


def _final_body(deg_ref, s_ref, g_ref, b_ref, wp_ref, bp_ref, o_ref):
    dinv = _dinv_from(deg_ref)
    s = s_ref[0, :N, :] + s_ref[1, :N, :] + g_ref[...]
    xn = jnp.maximum(dinv * s + b_ref[...], 0.0)
    o_ref[...] = jnp.dot(xn, wp_ref[...], preferred_element_type=jnp.float32) + bp_ref[...]


def _tc_call(body, out_dim):
    return pl.pallas_call(body, out_shape=jax.ShapeDtypeStruct((N, out_dim), jnp.float32))


def kernel(x, edge_index, W1, b1, W2, b2, W3, b3, Wp, bp):
    src = edge_index[0].astype(jnp.int32)
    dst = edge_index[1].astype(jnp.int32)
    pad = EPAD - E
    srcp = jnp.concatenate([src, jnp.zeros((pad,), jnp.int32)])
    dstp = jnp.concatenate([dst, jnp.full((pad,), N, jnp.int32)])
    src_r = srcp.reshape(NCORES, NSUB, G, J, R)
    dst_r = dstp.reshape(NCORES, NSUB, G, J, R)

    zeros64 = jnp.zeros((ROWS_PER_TILE, 64), jnp.float32)
    zeros32 = jnp.zeros((ROWS_PER_TILE, 32), jnp.float32)
    zeros16 = jnp.zeros((ROWS_PER_TILE, _DEG_D), jnp.float32)
    ones16 = jnp.ones((CHUNK, _DEG_D), jnp.float32)

    degP = _deg_kernel(dst_r, zeros16, ones16)

    g1 = _tc_call(_prep_body, 64)(degP, x, W1)
    S1 = _agg64(g1, src_r, dst_r, zeros64)
    g2 = _tc_call(_combine_body, 64)(degP, S1, g1, b1.reshape(1, -1), W2)
    S2 = _agg64(g2, src_r, dst_r, zeros64)
    g3 = _tc_call(_combine_body, 32)(degP, S2, g2, b2.reshape(1, -1), W3)
    S3 = _agg32(g3, src_r, dst_r, zeros32)
    y = _tc_call(_final_body, 1)(degP, S3, g3, b3.reshape(1, -1), Wp, bp.reshape(1, 1))
    return y
